# baseline (speedup 1.0000x reference)
import functools

import jax
import jax.numpy as jnp
from jax import lax
from jax.experimental import pallas as pl
from jax.experimental.pallas import tpu as pltpu

S = 2048
NH = 12
HD = 64
BLK = 32
SCB = S // BLK
WIN = 512
TOPK = 8
NEG = -1e30
QT = 256
HPP = 2


def _nsa_pair_kernel(q_ref, k_ref, v_ref, kc_ref, vc_ref, gw_ref, gc_ref, gs_ref, o_ref):
    scale = HD ** -0.5

    tcol = lax.broadcasted_iota(jnp.int32, (SCB, S), 1)
    brow = lax.broadcasted_iota(jnp.int32, (SCB, S), 0)
    memb = (tcol // BLK == brow).astype(jnp.bfloat16)

    qrow = lax.broadcasted_iota(jnp.int32, (S, SCB), 0)
    bcol = lax.broadcasted_iota(jnp.int32, (S, SCB), 1)
    cmask = (bcol + 1) * BLK - 1 <= qrow

    for t in range(HPP):
        lo, hi_ = t * HD, (t + 1) * HD
        qh = q_ref[:, lo:hi_]
        kh = k_ref[:, lo:hi_]
        vh = v_ref[:, lo:hi_]
        kc = kc_ref[:, lo:hi_]
        vc = vc_ref[:, lo:hi_]
        gw = jax.nn.sigmoid(gw_ref[t, 0])
        gc = jax.nn.sigmoid(gc_ref[t, 0])
        gs = jax.nn.sigmoid(gs_ref[t, 0])

        lc = jnp.dot(qh.astype(jnp.bfloat16), kc.astype(jnp.bfloat16).T,
                     preferred_element_type=jnp.float32) * scale
        lc_m = jnp.where(cmask, lc, NEG)

        work = lc_m
        mC = jnp.max(work, axis=-1, keepdims=True)
        m = mC
        for _ in range(TOPK - 1):
            work = jnp.where(work == m, NEG * 4.0, work)
            m = jnp.max(work, axis=-1, keepdims=True)
        sel_f = jnp.where((lc_m >= m) & cmask, 1.0, 0.0)

        pc = jnp.exp(lc_m - mC)
        any_c = mC > (NEG * 0.5)
        gcn = jnp.where(any_c[:, 0], gc, 0.0) / jnp.sum(pc, axis=-1)
        o_cmp = ((gcn[:, None] * pc) @ vc)

        qb = qh.astype(jnp.bfloat16)
        kb = kh.astype(jnp.bfloat16)
        vb = vh.astype(jnp.bfloat16)
        selb = sel_f.astype(jnp.bfloat16)
        for i in range(S // QT):
            qs = i * QT
            ke = (i + 1) * QT
            l = jnp.dot(qb[qs:qs + QT, :], kb[:ke, :].T,
                        preferred_element_type=jnp.float32) * scale

            rows = qs + lax.broadcasted_iota(jnp.int32, (QT, ke), 0)
            cols = lax.broadcasted_iota(jnp.int32, (QT, ke), 1)
            causal = rows >= cols
            win_f = jnp.where(causal & ((rows - cols) < WIN), 1.0, 0.0)

            sel_tok = jnp.dot(selb[qs:qs + QT, :], memb[:, :ke],
                              preferred_element_type=jnp.float32)
            m = jnp.max(l, axis=-1, keepdims=True)
            e = jnp.exp(l - m)
            ew = win_f * e
            es = sel_tok * e
            cw = gw[qs:qs + QT] / jnp.sum(ew, axis=-1)
            cs = gs[qs:qs + QT] / jnp.maximum(jnp.sum(es, axis=-1), 1e-30)
            p_comb = cw[:, None] * ew + cs[:, None] * es
            o_tile = jnp.dot(p_comb.astype(jnp.bfloat16), vb[:ke, :],
                             preferred_element_type=jnp.float32)
            o_ref[qs:qs + QT, lo:hi_] = o_tile + o_cmp[qs:qs + QT, :]


@functools.partial(jax.jit, static_argnames=())
def kernel(q, k, v, g_win, g_cmp, g_slt):
    q2 = q.reshape(S, NH * HD)
    k2 = k.reshape(S, NH * HD)
    v2 = v.reshape(S, NH * HD)
    kc2 = k.reshape(SCB, BLK, NH * HD).mean(axis=1)
    vc2 = v.reshape(SCB, BLK, NH * HD).mean(axis=1)
    gw = jnp.transpose(g_win, (1, 0)).reshape(NH, 1, S)
    gc = jnp.transpose(g_cmp, (1, 0)).reshape(NH, 1, S)
    gs = jnp.transpose(g_slt, (1, 0)).reshape(NH, 1, S)

    lane = pl.BlockSpec((S, HPP * HD), lambda g: (0, g))
    lanec = pl.BlockSpec((SCB, HPP * HD), lambda g: (0, g))
    sg = pl.BlockSpec((HPP, 1, S), lambda g: (g, 0, 0))
    o = pl.pallas_call(
        _nsa_pair_kernel,
        grid=(NH // HPP,),
        in_specs=[lane, lane, lane, lanec, lanec, sg, sg, sg],
        out_specs=lane,
        out_shape=jax.ShapeDtypeStruct((S, NH * HD), jnp.float32),
        compiler_params=pltpu.CompilerParams(
            dimension_semantics=("parallel",)),
    )(q2, k2, v2, kc2, vc2, gw, gc, gs)
    return o.reshape(S, NH, HD)


# speedup vs baseline: 1.0962x; 1.0962x over previous
import functools

import jax
import jax.numpy as jnp
from jax import lax
from jax.experimental import pallas as pl
from jax.experimental.pallas import tpu as pltpu

S = 2048
NH = 12
HD = 64
BLK = 32
SCB = S // BLK
WIN = 512
TOPK = 8
NEG = -1e30
QT = 256


def _nsa_head_kernel(q_ref, k_ref, v_ref, kc_ref, vc_ref, gw_ref, gc_ref, gs_ref, o_ref):
    qb = q_ref[0]
    kb = k_ref[0]
    vb = v_ref[0]
    kc = kc_ref[0]
    vc = vc_ref[0]
    gw = jax.nn.sigmoid(gw_ref[0, 0])
    gc = jax.nn.sigmoid(gc_ref[0, 0])
    gs = jax.nn.sigmoid(gs_ref[0, 0])
    scale = HD ** -0.5

    tcol = lax.broadcasted_iota(jnp.int32, (SCB, S), 1)
    brow = lax.broadcasted_iota(jnp.int32, (SCB, S), 0)
    memb = (tcol // BLK == brow).astype(jnp.bfloat16)

    lc = jnp.dot(qb, kc.astype(jnp.bfloat16).T,
                 preferred_element_type=jnp.float32) * scale
    qrow = lax.broadcasted_iota(jnp.int32, (S, SCB), 0)
    bcol = lax.broadcasted_iota(jnp.int32, (S, SCB), 1)
    cmask = (bcol + 1) * BLK - 1 <= qrow
    lc_m = jnp.where(cmask, lc, NEG)

    work = lc_m
    mC = jnp.max(work, axis=-1, keepdims=True)
    m = mC
    for _ in range(TOPK - 1):
        work = jnp.where(work == m, NEG * 4.0, work)
        m = jnp.max(work, axis=-1, keepdims=True)
    sel_f = jnp.where((lc_m >= m) & cmask, 1.0, 0.0)
    selb = sel_f.astype(jnp.bfloat16)

    pc = jnp.exp(lc_m - mC)
    any_c = mC > (NEG * 0.5)
    gcn = jnp.where(any_c[:, 0], gc, 0.0) / jnp.sum(pc, axis=-1)
    o_cmp = ((gcn[:, None] * pc) @ vc)

    for i in range(S // QT):
        qs = i * QT
        ke = (i + 1) * QT
        l = jnp.dot(qb[qs:qs + QT, :], kb[:ke, :].T,
                    preferred_element_type=jnp.float32) * scale

        rows = qs + lax.broadcasted_iota(jnp.int32, (QT, ke), 0)
        cols = lax.broadcasted_iota(jnp.int32, (QT, ke), 1)
        causal = rows >= cols
        win_f = jnp.where(causal & ((rows - cols) < WIN), 1.0, 0.0)

        sel_tok = jnp.dot(selb[qs:qs + QT, :], memb[:, :ke],
                          preferred_element_type=jnp.float32)
        m = jnp.max(l, axis=-1, keepdims=True)
        e = jnp.exp(l - m)
        ew = win_f * e
        es = sel_tok * e
        cw = gw[qs:qs + QT] / jnp.sum(ew, axis=-1)
        cs = gs[qs:qs + QT] / jnp.maximum(jnp.sum(es, axis=-1), 1e-30)
        p_comb = cw[:, None] * ew + cs[:, None] * es
        o_tile = jnp.dot(p_comb.astype(jnp.bfloat16), vb[:ke, :],
                         preferred_element_type=jnp.float32)
        o_ref[0, qs:qs + QT, :] = o_tile + o_cmp[qs:qs + QT, :]


@functools.partial(jax.jit, static_argnames=())
def kernel(q, k, v, g_win, g_cmp, g_slt):
    qh = jnp.transpose(q.astype(jnp.bfloat16), (1, 0, 2))
    kh = jnp.transpose(k.astype(jnp.bfloat16), (1, 0, 2))
    vh = jnp.transpose(v.astype(jnp.bfloat16), (1, 0, 2))
    kc = jnp.transpose(k.reshape(SCB, BLK, NH, HD).mean(axis=1), (1, 0, 2))
    vc = jnp.transpose(v.reshape(SCB, BLK, NH, HD).mean(axis=1), (1, 0, 2))
    gw = jnp.transpose(g_win, (1, 0)).reshape(NH, 1, S)
    gc = jnp.transpose(g_cmp, (1, 0)).reshape(NH, 1, S)
    gs = jnp.transpose(g_slt, (1, 0)).reshape(NH, 1, S)

    shd = pl.BlockSpec((1, S, HD), lambda h: (h, 0, 0))
    sc = pl.BlockSpec((1, SCB, HD), lambda h: (h, 0, 0))
    sg = pl.BlockSpec((1, 1, S), lambda h: (h, 0, 0))
    o = pl.pallas_call(
        _nsa_head_kernel,
        grid=(NH,),
        in_specs=[shd, shd, shd, sc, sc, sg, sg, sg],
        out_specs=shd,
        out_shape=jax.ShapeDtypeStruct((NH, S, HD), jnp.float32),
        compiler_params=pltpu.CompilerParams(
            dimension_semantics=("parallel",)),
    )(qh, kh, vh, kc, vc, gw, gc, gs)
    return jnp.transpose(o, (1, 0, 2))


# speedup vs baseline: 1.8852x; 1.7197x over previous
import functools

import jax
import jax.numpy as jnp
from jax import lax
from jax.experimental import pallas as pl
from jax.experimental.pallas import tpu as pltpu

S = 2048
NH = 12
HD = 64
BLK = 32
SCB = S // BLK
WIN = 512
TOPK = 8
NEG = -1e30
QT = 256


def _nsa_head_kernel(q_ref, k_ref, v_ref, kc_ref, vc_ref, gw_ref, gc_ref, gs_ref, o_ref):
    qh = q_ref[0]
    qb = qh.astype(jnp.bfloat16)
    kb = k_ref[0].astype(jnp.bfloat16)
    vb = v_ref[0].astype(jnp.bfloat16)
    kc = kc_ref[0]
    vc = vc_ref[0]
    gw = jax.nn.sigmoid(gw_ref[0, 0])
    gc = jax.nn.sigmoid(gc_ref[0, 0])
    gs = jax.nn.sigmoid(gs_ref[0, 0])
    scale = HD ** -0.5

    vaug = jnp.concatenate(
        [vb, jnp.ones((S, 1), dtype=jnp.bfloat16)], axis=1)

    tcol = lax.broadcasted_iota(jnp.int32, (SCB, S), 1)
    brow = lax.broadcasted_iota(jnp.int32, (SCB, S), 0)
    memb = (tcol // BLK == brow).astype(jnp.bfloat16)

    lcT = lax.dot_general(kc.astype(jnp.bfloat16), qb,
                          (((1,), (1,)), ((), ())),
                          preferred_element_type=jnp.float32) * scale
    qcol = lax.broadcasted_iota(jnp.int32, (SCB, S), 1)
    cmaskT = (brow + 1) * BLK - 1 <= qcol
    lcTm = jnp.where(cmaskT, lcT, NEG)

    work = lcTm
    mC = jnp.max(work, axis=0, keepdims=True)
    m = mC
    for _ in range(TOPK - 1):
        work = jnp.where(work == m, NEG * 4.0, work)
        m = jnp.max(work, axis=0, keepdims=True)
    selT = jnp.where((lcTm >= m) & cmaskT, 1.0, 0.0).astype(jnp.bfloat16)

    pcT = jnp.exp(lcTm - mC)
    any_c = mC[0] > (NEG * 0.5)
    gcn = jnp.where(any_c, gc, 0.0) / jnp.sum(pcT, axis=0)
    o_cmp = lax.dot_general(pcT * gcn[None, :], vc,
                            (((0,), (0,)), ((), ())),
                            preferred_element_type=jnp.float32)

    for i in range(S // QT):
        qs = i * QT
        ke = (i + 1) * QT
        l = jnp.dot(qb[qs:qs + QT, :], kb[:ke, :].T,
                    preferred_element_type=jnp.float32) * scale

        rows = qs + lax.broadcasted_iota(jnp.int32, (QT, ke), 0)
        cols = lax.broadcasted_iota(jnp.int32, (QT, ke), 1)
        win_f = jnp.where((rows - cols).astype(jnp.uint32) < WIN, 1.0, 0.0)

        sel_tok = lax.dot_general(selT[:, qs:qs + QT], memb[:, :ke],
                                  (((0,), (0,)), ((), ())),
                                  preferred_element_type=jnp.float32)
        m = jnp.max(l, axis=-1, keepdims=True)
        e = jnp.exp(l - m)
        ewb = (win_f * e).astype(jnp.bfloat16)
        esb = (sel_tok * e).astype(jnp.bfloat16)
        ow = jnp.dot(ewb, vaug[:ke, :], preferred_element_type=jnp.float32)
        os_ = jnp.dot(esb, vaug[:ke, :], preferred_element_type=jnp.float32)
        cw = gw[qs:qs + QT] / jnp.maximum(ow[:, HD], 1e-30)
        cs = gs[qs:qs + QT] / jnp.maximum(os_[:, HD], 1e-30)
        o_tile = (cw[:, None] * ow[:, :HD] + cs[:, None] * os_[:, :HD]
                  + o_cmp[qs:qs + QT, :])
        o_ref[0, qs:qs + QT, :] = o_tile


@functools.partial(jax.jit, static_argnames=())
def kernel(q, k, v, g_win, g_cmp, g_slt):
    qh = jnp.transpose(q, (1, 0, 2))
    kh = jnp.transpose(k, (1, 0, 2))
    vh = jnp.transpose(v, (1, 0, 2))
    kc = kh.reshape(NH, SCB, BLK, HD).mean(axis=2)
    vc = vh.reshape(NH, SCB, BLK, HD).mean(axis=2)
    gw = jnp.transpose(g_win, (1, 0)).reshape(NH, 1, S)
    gc = jnp.transpose(g_cmp, (1, 0)).reshape(NH, 1, S)
    gs = jnp.transpose(g_slt, (1, 0)).reshape(NH, 1, S)

    shd = pl.BlockSpec((1, S, HD), lambda h: (h, 0, 0))
    sc = pl.BlockSpec((1, SCB, HD), lambda h: (h, 0, 0))
    sg = pl.BlockSpec((1, 1, S), lambda h: (h, 0, 0))
    o = pl.pallas_call(
        _nsa_head_kernel,
        grid=(NH,),
        in_specs=[shd, shd, shd, sc, sc, sg, sg, sg],
        out_specs=shd,
        out_shape=jax.ShapeDtypeStruct((NH, S, HD), jnp.float32),
        compiler_params=pltpu.CompilerParams(
            dimension_semantics=("parallel",)),
    )(qh, kh, vh, kc, vc, gw, gc, gs)
    return jnp.transpose(o, (1, 0, 2))


# speedup vs baseline: 2.0325x; 1.0781x over previous
import functools

import jax
import jax.numpy as jnp
from jax import lax
from jax.experimental import pallas as pl
from jax.experimental.pallas import tpu as pltpu

S = 2048
NH = 12
HD = 64
BLK = 32
SCB = S // BLK
WIN = 512
TOPK = 8
NEG = -1e30
QT = 256


def _nsa_head_kernel(q_ref, k_ref, v_ref, kc_ref, vc_ref, g_ref, o_ref):
    qh = q_ref[0]
    qb = qh.astype(jnp.bfloat16)
    kb = k_ref[0].astype(jnp.bfloat16)
    vb = v_ref[0].astype(jnp.bfloat16)
    kc = kc_ref[0]
    vc = vc_ref[0]
    gw = jax.nn.sigmoid(g_ref[0, 0, 0])
    gc = jax.nn.sigmoid(g_ref[1, 0, 0])
    gs = jax.nn.sigmoid(g_ref[2, 0, 0])
    scale = HD ** -0.5

    vaug = jnp.concatenate(
        [vb, jnp.ones((S, 1), dtype=jnp.bfloat16)], axis=1)

    tcol = lax.broadcasted_iota(jnp.int32, (SCB, S), 1)
    brow = lax.broadcasted_iota(jnp.int32, (SCB, S), 0)
    memb = (tcol // BLK == brow).astype(jnp.bfloat16)

    lcT = lax.dot_general(kc.astype(jnp.bfloat16), qb,
                          (((1,), (1,)), ((), ())),
                          preferred_element_type=jnp.float32) * scale
    cmaskT = (brow + 1) * BLK - 1 <= tcol
    lcTm = jnp.where(cmaskT, lcT, NEG)

    work = lcTm
    mC = jnp.max(work, axis=0, keepdims=True)
    m = mC
    for _ in range(TOPK - 1):
        work = jnp.where(work == m, NEG * 4.0, work)
        m = jnp.max(work, axis=0, keepdims=True)
    selT = jnp.where((lcTm >= m) & cmaskT, 1.0, 0.0).astype(jnp.bfloat16)

    pcT = jnp.exp(lcTm - mC)
    any_c = mC[0] > (NEG * 0.5)
    gcn = jnp.where(any_c, gc, 0.0) / jnp.sum(pcT, axis=0)
    o_cmp = lax.dot_general(pcT * gcn[None, :], vc,
                            (((0,), (0,)), ((), ())),
                            preferred_element_type=jnp.float32)

    for i in range(S // QT):
        qs = i * QT
        ke = (i + 1) * QT
        l = jnp.dot(qb[qs:qs + QT, :], kb[:ke, :].T,
                    preferred_element_type=jnp.float32) * scale

        sb = max(0, qs - WIN)
        W = ke - sb
        rows = qs + lax.broadcasted_iota(jnp.int32, (QT, W), 0)
        cols = sb + lax.broadcasted_iota(jnp.int32, (QT, W), 1)
        win_b = jnp.where((rows - cols).astype(jnp.uint32) < WIN,
                          1.0, 0.0).astype(jnp.bfloat16)

        sel_tok = lax.dot_general(selT[:, qs:qs + QT], memb[:, :ke],
                                  (((0,), (0,)), ((), ())),
                                  preferred_element_type=jnp.float32)
        m = jnp.max(l, axis=-1, keepdims=True)
        eb = jnp.exp(l - m).astype(jnp.bfloat16)
        ewb = win_b * eb[:, sb:ke]
        esb = sel_tok.astype(jnp.bfloat16) * eb
        ow = jnp.dot(ewb, vaug[sb:ke, :], preferred_element_type=jnp.float32)
        os_ = jnp.dot(esb, vaug[:ke, :], preferred_element_type=jnp.float32)
        cw = gw[qs:qs + QT] / jnp.maximum(ow[:, HD], 1e-30)
        cs = gs[qs:qs + QT] / jnp.maximum(os_[:, HD], 1e-30)
        o_tile = (cw[:, None] * ow[:, :HD] + cs[:, None] * os_[:, :HD]
                  + o_cmp[qs:qs + QT, :])
        o_ref[0, qs:qs + QT, :] = o_tile


@functools.partial(jax.jit, static_argnames=())
def kernel(q, k, v, g_win, g_cmp, g_slt):
    qh = jnp.transpose(q, (1, 0, 2))
    kh = jnp.transpose(k, (1, 0, 2))
    vh = jnp.transpose(v, (1, 0, 2))
    kc = kh.reshape(NH, SCB, BLK, HD).mean(axis=2)
    vc = vh.reshape(NH, SCB, BLK, HD).mean(axis=2)
    gall = jnp.transpose(jnp.stack([g_win, g_cmp, g_slt]), (0, 2, 1))
    gall = gall.reshape(3, NH, 1, S)

    shd = pl.BlockSpec((1, S, HD), lambda h: (h, 0, 0))
    sc = pl.BlockSpec((1, SCB, HD), lambda h: (h, 0, 0))
    sg = pl.BlockSpec((3, 1, 1, S), lambda h: (0, h, 0, 0))
    o = pl.pallas_call(
        _nsa_head_kernel,
        grid=(NH,),
        in_specs=[shd, shd, shd, sc, sc, sg],
        out_specs=shd,
        out_shape=jax.ShapeDtypeStruct((NH, S, HD), jnp.float32),
        compiler_params=pltpu.CompilerParams(
            dimension_semantics=("parallel",)),
    )(qh, kh, vh, kc, vc, gall)
    return jnp.transpose(o, (1, 0, 2))
